# baseline (speedup 1.0000x reference)
import functools
import math

import jax
import jax.numpy as jnp
from jax import lax
from jax.experimental import pallas as pl
from jax.experimental.pallas import tpu as pltpu
from jax.experimental.pallas import tpu_sc as plsc

N = 10000
E = 320000
F = 128
AVG_LOG = math.log(33.0)
BLK = 2000

NS = 16
NT = 32
RPT = 320
HRPT = 160
NPAD = NT * RPT
CHUNK = 1280
NCHUNK = E // CHUNK
BATCH = 64
TFLUSH = 1024
LCAP = 2432
STRASH = NS * HRPT


def _pre_body(x_ref, wcat_ref, bpre_ref, out_ref):
    x = x_ref[...]
    w = wcat_ref[...]
    out = jax.lax.dot_general(x, w, (((1,), (0,)), ((), ())),
                              preferred_element_type=jnp.float32,
                              precision=jax.lax.Precision.HIGHEST)
    out = out + jnp.concatenate([bpre_ref[...], jnp.zeros((1, F), jnp.float32)],
                                axis=1)
    out_ref[...] = out


def _pre(x, W_pre, b_pre):
    wcat = jnp.concatenate([W_pre[:, :F].T, W_pre[:, F:].T], axis=1)
    return pl.pallas_call(
        _pre_body,
        grid=(N // BLK,),
        in_specs=[
            pl.BlockSpec((BLK, F), lambda i: (i, 0)),
            pl.BlockSpec((F, 2 * F), lambda i: (0, 0)),
            pl.BlockSpec((1, F), lambda i: (0, 0)),
        ],
        out_specs=pl.BlockSpec((BLK, 2 * F), lambda i: (i, 0)),
        out_shape=jax.ShapeDtypeStruct((N, 2 * F), jnp.float32),
    )(x, wcat, b_pre.reshape(1, F))


def _sc_segment_stats(dst, src, B):
    mesh = plsc.VectorSubcoreMesh(core_axis_name="c", subcore_axis_name="s")
    out_type = (
        jax.ShapeDtypeStruct((NT, 320), jnp.int32),
        jax.ShapeDtypeStruct((NPAD, F), jnp.float32),
        jax.ShapeDtypeStruct((NPAD, F), jnp.float32),
        jax.ShapeDtypeStruct((NPAD, F), jnp.float32),
        jax.ShapeDtypeStruct((NPAD, F), jnp.float32),
    )
    scratch = [
        pltpu.VMEM((2 * CHUNK,), jnp.int32),
        pltpu.VMEM((2 * CHUNK,), jnp.int32),
        pltpu.VMEM((LCAP,), jnp.int32),
        pltpu.VMEM((LCAP,), jnp.int32),
        pltpu.VMEM((BATCH,), jnp.int32),
        pltpu.VMEM((BATCH,), jnp.int32),
        pltpu.VMEM((BATCH + 16,), jnp.int32),
        pltpu.VMEM((BATCH, F), jnp.float32),
        pltpu.VMEM((BATCH, F), jnp.float32),
        pltpu.VMEM((BATCH,), jnp.int32),
        pltpu.VMEM((BATCH,), jnp.int32),
        pltpu.VMEM((BATCH + 16,), jnp.int32),
        pltpu.VMEM((BATCH, F), jnp.float32),
        pltpu.VMEM((BATCH, F), jnp.float32),
        pltpu.VMEM((HRPT + 1, F), jnp.float32),
        pltpu.VMEM((HRPT + 1, F), jnp.float32),
        pltpu.VMEM((RPT,), jnp.int32),
        pltpu.VMEM_SHARED((STRASH + 1, F), jnp.float32),
        pltpu.VMEM_SHARED((STRASH + 1, F), jnp.float32),
        pltpu.SemaphoreType.DMA,
        pltpu.SemaphoreType.DMA,
        pltpu.SemaphoreType.DMA,
        pltpu.SemaphoreType.DMA,
    ]

    @functools.partial(
        pl.kernel, out_type=out_type, mesh=mesh, scratch_types=scratch,
        compiler_params=pltpu.CompilerParams(needs_layout_passes=False))
    def sck(dst_hbm, src_hbm, b_hbm, cnt_out, sum_out, sum2_out, mn_out, mx_out,
            dbuf, sbuf, mlist_s, mlist_d,
            sidxA, slocalA, dlbufA, rowsA, sqbufA,
            sidxB, slocalB, dlbufB, rowsB, sqbufB,
            mn_acc, mx_acc, cntv, ssum, ssum2, sem_e, sem_g, sem_s, sem_s2):
        bufsA = (sidxA, slocalA, dlbufA, rowsA, sqbufA)
        bufsB = (sidxB, slocalB, dlbufB, rowsB, sqbufB)
        cc = lax.axis_index("c")
        ss = lax.axis_index("s")
        wid = cc * NS + ss
        lo_t = wid * RPT
        sbase = ss * HRPT

        pinf = jnp.full((16,), jnp.inf, jnp.float32)
        ninf = jnp.full((16,), -jnp.inf, jnp.float32)
        zf = jnp.zeros((16,), jnp.float32)
        zi = jnp.zeros((16,), jnp.int32)

        for kk in range(RPT // 16):
            cntv[pl.ds(kk * 16, 16)] = zi

        for hp in range(2):
            lo = lo_t + hp * HRPT

            def init_acc(i, carry):
                for c8 in range(8):
                    mn_acc[i, pl.ds(c8 * 16, 16)] = pinf
                    mx_acc[i, pl.ds(c8 * 16, 16)] = ninf
                return carry
            lax.fori_loop(0, HRPT + 1, init_acc, 0)

            def zrow(i, carry):
                for c8 in range(8):
                    rowsA[i, pl.ds(c8 * 16, 16)] = zf
                return carry
            lax.fori_loop(0, BATCH, zrow, 0)
            for tgt in (ssum, ssum2):
                pltpu.sync_copy(rowsA, tgt.at[pl.ds(sbase, BATCH)])
                pltpu.sync_copy(rowsA, tgt.at[pl.ds(sbase + BATCH, BATCH)])
                pltpu.sync_copy(rowsA.at[pl.ds(0, HRPT - 2 * BATCH)],
                                tgt.at[pl.ds(sbase + 2 * BATCH,
                                             HRPT - 2 * BATCH)])

            def fill_and_fire(off, bufs):
                sidx, slocal, dlbuf, rows, _ = bufs
                for kk in range(BATCH // 16):
                    sidx[pl.ds(kk * 16, 16)] = mlist_s[pl.ds(off + kk * 16, 16)]
                    v = mlist_d[pl.ds(off + kk * 16, 16)]
                    slocal[pl.ds(kk * 16, 16)] = v
                    dlbuf[pl.ds(kk * 16, 16)] = jnp.minimum(v - sbase, HRPT)
                pltpu.async_copy(b_hbm.at[sidx], rows, sem_g)

            def wait_sum2(bufs):
                _, slocal, _, _, sqbuf = bufs
                pltpu.make_async_copy(sqbuf, ssum2.at[slocal], sem_s2).wait()

            def do_batch(b, nb, cur, oth):
                sidx, slocal, dlbuf, rows, sqbuf = cur
                pltpu.make_async_copy(b_hbm.at[sidx], rows, sem_g).wait()
                pltpu.async_copy(rows, ssum.at[slocal], sem_s, add=True)

                @pl.when(b + 1 < nb)
                def _():
                    @pl.when(b >= 1)
                    def _():
                        wait_sum2(oth)
                    fill_and_fire((b + 1) * BATCH, oth)

                def sq_loop(i, carry2):
                    for c8 in range(8):
                        cs = pl.ds(c8 * 16, 16)
                        r = rows[i, cs]
                        sqbuf[i, cs] = r * r
                    return carry2
                lax.fori_loop(0, BATCH, sq_loop, 0, unroll=4)

                def edge_body(i, carry2):
                    dl = dlbuf[pl.ds(i, 16)][0]
                    for c8 in range(8):
                        cs = pl.ds(c8 * 16, 16)
                        r = rows[i, cs]
                        mn_acc[dl, cs] = jnp.minimum(mn_acc[dl, cs], r)
                        mx_acc[dl, cs] = jnp.maximum(mx_acc[dl, cs], r)
                    return carry2
                lax.fori_loop(0, BATCH, edge_body, 0, unroll=4)
                pltpu.make_async_copy(rows, ssum.at[slocal], sem_s).wait()
                pltpu.async_copy(sqbuf, ssum2.at[slocal], sem_s2, add=True)

            def flush(nb):
                @pl.when(nb > 0)
                def _():
                    fill_and_fire(0, bufsA)

                def bloop(b, carry):
                    par2 = lax.rem(b, 2)

                    @pl.when(par2 == 0)
                    def _():
                        do_batch(b, nb, bufsA, bufsB)

                    @pl.when(par2 == 1)
                    def _():
                        do_batch(b, nb, bufsB, bufsA)
                    return carry
                lax.fori_loop(0, nb, bloop, 0)

                @pl.when(nb >= 2)
                def _():
                    @pl.when(lax.rem(nb, 2) == 0)
                    def _():
                        wait_sum2(bufsA)
                    @pl.when(lax.rem(nb, 2) == 1)
                    def _():
                        wait_sum2(bufsB)

                @pl.when(nb >= 1)
                def _():
                    @pl.when(lax.rem(nb, 2) == 1)
                    def _():
                        wait_sum2(bufsA)
                    @pl.when(lax.rem(nb, 2) == 0)
                    def _():
                        wait_sum2(bufsB)

            pltpu.async_copy(dst_hbm.at[pl.ds(0, CHUNK)],
                             dbuf.at[pl.ds(0, CHUNK)], sem_e)
            pltpu.async_copy(src_hbm.at[pl.ds(0, CHUNK)],
                             sbuf.at[pl.ds(0, CHUNK)], sem_e)

            def chunk_body(k, pos):
                par = lax.rem(k, 2)
                off0 = par * CHUNK
                pltpu.make_async_copy(dst_hbm.at[pl.ds(k * CHUNK, CHUNK)],
                                      dbuf.at[pl.ds(off0, CHUNK)], sem_e).wait()
                pltpu.make_async_copy(src_hbm.at[pl.ds(k * CHUNK, CHUNK)],
                                      sbuf.at[pl.ds(off0, CHUNK)], sem_e).wait()

                @pl.when(k < NCHUNK - 1)
                def _():
                    noff = (1 - par) * CHUNK
                    pltpu.async_copy(dst_hbm.at[pl.ds((k + 1) * CHUNK, CHUNK)],
                                     dbuf.at[pl.ds(noff, CHUNK)], sem_e)
                    pltpu.async_copy(src_hbm.at[pl.ds((k + 1) * CHUNK, CHUNK)],
                                     sbuf.at[pl.ds(noff, CHUNK)], sem_e)

                ones16 = jnp.ones((16,), jnp.int32)

                def scan_body(jj, p):
                    base = off0 + jj * 128
                    gs = []
                    for g in range(8):
                        d = dbuf[pl.ds(base + g * 16, 16)]
                        sv = sbuf[pl.ds(base + g * 16, 16)]
                        m = (d >= lo) & (d < lo + HRPT)
                        cs = plsc.cumsum(m.astype(jnp.int32))
                        gs.append((d, sv, m, cs, cs[15]))
                    pg = p
                    for d, sv, m, cs, cnt in gs:
                        posv = pg + cs - 1
                        plsc.store_scatter(mlist_d, [posv], d - (lo - sbase),
                                           mask=m)
                        plsc.store_scatter(mlist_s, [posv], sv, mask=m)
                        plsc.addupdate_scatter(cntv, [d - lo_t], ones16, mask=m)
                        pg = pg + cnt
                    return pg
                pos = lax.fori_loop(0, CHUNK // 128, scan_body, pos)

                nb = jnp.where(pos >= TFLUSH, pos // BATCH, 0)
                flush(nb)
                rem = pos - nb * BATCH

                @pl.when(nb > 0)
                def _():
                    for kk in range(8):
                        vd = mlist_d[pl.ds(nb * BATCH + kk * 16, 16)]
                        vs = mlist_s[pl.ds(nb * BATCH + kk * 16, 16)]
                        mlist_d[pl.ds(kk * 16, 16)] = vd
                        mlist_s[pl.ds(kk * 16, 16)] = vs
                return rem

            pos = lax.fori_loop(0, NCHUNK, chunk_body, jnp.int32(0))

            @pl.when(pos > 0)
            def _():
                fb = (pos // BATCH) * BATCH
                for kk in range(BATCH // 16):
                    gi = lax.iota(jnp.int32, 16) + (kk * 16) + fb
                    keep = gi < pos
                    vs = mlist_s[pl.ds(fb + kk * 16, 16)]
                    mlist_s[pl.ds(fb + kk * 16, 16)] = jnp.where(keep, vs, 0)
                    vd = mlist_d[pl.ds(fb + kk * 16, 16)]
                    mlist_d[pl.ds(fb + kk * 16, 16)] = jnp.where(keep, vd,
                                                                 STRASH)
                flush((pos + BATCH - 1) // BATCH)

            pltpu.sync_copy(mn_acc.at[pl.ds(0, HRPT)],
                            mn_out.at[pl.ds(lo, HRPT)])
            pltpu.sync_copy(mx_acc.at[pl.ds(0, HRPT)],
                            mx_out.at[pl.ds(lo, HRPT)])
            pltpu.sync_copy(ssum.at[pl.ds(sbase, HRPT)],
                            sum_out.at[pl.ds(lo, HRPT)])
            pltpu.sync_copy(ssum2.at[pl.ds(sbase, HRPT)],
                            sum2_out.at[pl.ds(lo, HRPT)])

        pltpu.sync_copy(cntv, cnt_out.at[wid])

    return sck(dst, src, B)


def _post_body(x_ref, a_ref, cnt_ref, sb_ref, sb2_ref, mn_ref, mx_ref,
               wp_ref, bp_ref, wl_ref, bl_ref, out_ref):
    x = x_ref[...]
    a = a_ref[...]
    cnt = cnt_ref[...].astype(jnp.float32)
    cntc = jnp.maximum(cnt, 1.0)
    inv = 1.0 / cntc
    has = cnt > 0.0
    frac = cnt * inv
    sb = sb_ref[...]
    mb = sb * inv
    mb2 = sb2_ref[...] * inv
    mean = a * frac + mb
    var = jax.nn.relu(mb2 - mb * mb)
    std = jnp.sqrt(var + 1e-5)
    mn = jnp.where(has, a + mn_ref[...], 0.0)
    mx = jnp.where(has, a + mx_ref[...], 0.0)
    logd = jnp.log(cntc + 1.0)
    amp = logd * (1.0 / AVG_LOG)
    att = AVG_LOG / logd
    parts = [x, mean, mn, mx, std,
             mean * amp, mn * amp, mx * amp, std * amp,
             mean * att, mn * att, mx * att, std * att]
    wp = wp_ref[...]
    dn = (((1,), (0,)), ((), ()))
    out1 = bp_ref[...].astype(jnp.float32)
    for i, p in enumerate(parts):
        out1 = out1 + jax.lax.dot_general(
            p, wp[i * F:(i + 1) * F, :], dn,
            preferred_element_type=jnp.float32,
            precision=jax.lax.Precision.HIGHEST)
    out2 = jax.lax.dot_general(out1, wl_ref[...], dn,
                               preferred_element_type=jnp.float32,
                               precision=jax.lax.Precision.HIGHEST)
    out_ref[...] = out2 + bl_ref[...]


def _post(x, A, cnt, sB, sB2, mnB, mxB, W_post, b_post, W_lin, b_lin):
    wp = W_post.T
    wl = W_lin.T
    blk = lambda i: (i, 0)
    full = lambda i: (0, 0)
    return pl.pallas_call(
        _post_body,
        grid=(N // BLK,),
        in_specs=[
            pl.BlockSpec((BLK, F), blk),
            pl.BlockSpec((BLK, F), blk),
            pl.BlockSpec((BLK, 1), blk),
            pl.BlockSpec((BLK, F), blk),
            pl.BlockSpec((BLK, F), blk),
            pl.BlockSpec((BLK, F), blk),
            pl.BlockSpec((BLK, F), blk),
            pl.BlockSpec((13 * F, F), full),
            pl.BlockSpec((1, F), full),
            pl.BlockSpec((F, F), full),
            pl.BlockSpec((1, F), full),
        ],
        out_specs=pl.BlockSpec((BLK, F), blk),
        out_shape=jax.ShapeDtypeStruct((N, F), jnp.float32),
    )(x, A, cnt.reshape(N, 1), sB, sB2, mnB, mxB,
      wp, b_post.reshape(1, F), wl, b_lin.reshape(1, F))


def kernel(x, edge_index, W_pre, b_pre, W_post, b_post, W_lin, b_lin):
    src = edge_index[0]
    dst = edge_index[1]
    AB = _pre(x, W_pre, b_pre)
    A = AB[:, :F]
    B = AB[:, F:]
    cnt2d, sB, sB2, mnB, mxB = _sc_segment_stats(dst, src, B)
    cnt = cnt2d.reshape(NPAD)[:N]
    return _post(x, A, cnt, sB[:N], sB2[:N], mnB[:N], mxB[:N],
                 W_post, b_post, W_lin, b_lin)


# speedup vs baseline: 1.0307x; 1.0307x over previous
import functools
import math

import jax
import jax.numpy as jnp
from jax import lax
from jax.experimental import pallas as pl
from jax.experimental.pallas import tpu as pltpu
from jax.experimental.pallas import tpu_sc as plsc

N = 10000
E = 320000
F = 128
AVG_LOG = math.log(33.0)
BLK = 2000

NS = 16
NT = 32
RPT = 320
HRPT = 160
NPAD = NT * RPT
CHUNK = 1280
NCHUNK = E // CHUNK
BATCH = 64
TFLUSH = 1024
LCAP = 2432
STRASH = NS * HRPT


def _pre_body(x_ref, wcat_ref, bpre_ref, out_ref):
    x = x_ref[...]
    w = wcat_ref[...]
    out = jax.lax.dot_general(x, w, (((1,), (0,)), ((), ())),
                              preferred_element_type=jnp.float32,
                              precision=jax.lax.Precision.HIGHEST)
    out = out + jnp.concatenate([bpre_ref[...], jnp.zeros((1, F), jnp.float32)],
                                axis=1)
    out_ref[...] = out


def _pre(x, W_pre, b_pre):
    wcat = jnp.concatenate([W_pre[:, :F].T, W_pre[:, F:].T], axis=1)
    return pl.pallas_call(
        _pre_body,
        grid=(N // BLK,),
        in_specs=[
            pl.BlockSpec((BLK, F), lambda i: (i, 0)),
            pl.BlockSpec((F, 2 * F), lambda i: (0, 0)),
            pl.BlockSpec((1, F), lambda i: (0, 0)),
        ],
        out_specs=pl.BlockSpec((BLK, 2 * F), lambda i: (i, 0)),
        out_shape=jax.ShapeDtypeStruct((N, 2 * F), jnp.float32),
    )(x, wcat, b_pre.reshape(1, F))


def _sc_segment_stats(dst, src, B):
    mesh = plsc.VectorSubcoreMesh(core_axis_name="c", subcore_axis_name="s")
    out_type = (
        jax.ShapeDtypeStruct((NT, 320), jnp.int32),
        jax.ShapeDtypeStruct((NPAD, F), jnp.float32),
        jax.ShapeDtypeStruct((NPAD, F), jnp.float32),
        jax.ShapeDtypeStruct((NPAD, F), jnp.float32),
        jax.ShapeDtypeStruct((NPAD, F), jnp.float32),
    )
    scratch = [
        pltpu.VMEM((2 * CHUNK,), jnp.int32),
        pltpu.VMEM((2 * CHUNK,), jnp.int32),
        pltpu.VMEM((LCAP,), jnp.int32),
        pltpu.VMEM((LCAP,), jnp.int32),
        pltpu.VMEM((BATCH,), jnp.int32),
        pltpu.VMEM((BATCH,), jnp.int32),
        pltpu.VMEM((BATCH + 16,), jnp.int32),
        pltpu.VMEM((BATCH, F), jnp.float32),
        pltpu.VMEM((BATCH, F), jnp.float32),
        pltpu.VMEM((BATCH,), jnp.int32),
        pltpu.VMEM((BATCH,), jnp.int32),
        pltpu.VMEM((BATCH + 16,), jnp.int32),
        pltpu.VMEM((BATCH, F), jnp.float32),
        pltpu.VMEM((BATCH, F), jnp.float32),
        pltpu.VMEM((HRPT + 1, F), jnp.float32),
        pltpu.VMEM((HRPT + 1, F), jnp.float32),
        pltpu.VMEM((RPT,), jnp.int32),
        pltpu.VMEM_SHARED((STRASH + 1, F), jnp.float32),
        pltpu.VMEM_SHARED((STRASH + 1, F), jnp.float32),
        pltpu.SemaphoreType.DMA,
        pltpu.SemaphoreType.DMA,
        pltpu.SemaphoreType.DMA,
        pltpu.SemaphoreType.DMA,
    ]

    @functools.partial(
        pl.kernel, out_type=out_type, mesh=mesh, scratch_types=scratch,
        compiler_params=pltpu.CompilerParams(needs_layout_passes=False))
    def sck(dst_hbm, src_hbm, b_hbm, cnt_out, sum_out, sum2_out, mn_out, mx_out,
            dbuf, sbuf, mlist_s, mlist_d,
            sidxA, slocalA, dlbufA, rowsA, sqbufA,
            sidxB, slocalB, dlbufB, rowsB, sqbufB,
            mn_acc, mx_acc, cntv, ssum, ssum2, sem_e, sem_g, sem_s, sem_s2):
        bufsA = (sidxA, slocalA, dlbufA, rowsA, sqbufA)
        bufsB = (sidxB, slocalB, dlbufB, rowsB, sqbufB)
        cc = lax.axis_index("c")
        ss = lax.axis_index("s")
        wid = cc * NS + ss
        lo_t = wid * RPT
        sbase = ss * HRPT

        pinf = jnp.full((16,), jnp.inf, jnp.float32)
        ninf = jnp.full((16,), -jnp.inf, jnp.float32)
        zf = jnp.zeros((16,), jnp.float32)
        zi = jnp.zeros((16,), jnp.int32)

        for kk in range(RPT // 16):
            cntv[pl.ds(kk * 16, 16)] = zi

        for hp in range(2):
            lo = lo_t + hp * HRPT

            def init_acc(i, carry):
                for c8 in range(8):
                    mn_acc[i, pl.ds(c8 * 16, 16)] = pinf
                    mx_acc[i, pl.ds(c8 * 16, 16)] = ninf
                return carry
            lax.fori_loop(0, HRPT + 1, init_acc, 0)

            def zrow(i, carry):
                for c8 in range(8):
                    rowsA[i, pl.ds(c8 * 16, 16)] = zf
                return carry
            lax.fori_loop(0, BATCH, zrow, 0)
            for tgt in (ssum, ssum2):
                pltpu.sync_copy(rowsA, tgt.at[pl.ds(sbase, BATCH)])
                pltpu.sync_copy(rowsA, tgt.at[pl.ds(sbase + BATCH, BATCH)])
                pltpu.sync_copy(rowsA.at[pl.ds(0, HRPT - 2 * BATCH)],
                                tgt.at[pl.ds(sbase + 2 * BATCH,
                                             HRPT - 2 * BATCH)])

            def fill_and_fire(off, bufs):
                sidx, slocal, dlbuf, rows, _ = bufs
                for kk in range(BATCH // 16):
                    sidx[pl.ds(kk * 16, 16)] = mlist_s[pl.ds(off + kk * 16, 16)]
                    v = mlist_d[pl.ds(off + kk * 16, 16)]
                    slocal[pl.ds(kk * 16, 16)] = v
                    dlbuf[pl.ds(kk * 16, 16)] = jnp.minimum(v - sbase, HRPT)
                pltpu.async_copy(b_hbm.at[sidx], rows, sem_g)

            def wait_sum2(bufs):
                _, slocal, _, _, sqbuf = bufs
                pltpu.make_async_copy(sqbuf, ssum2.at[slocal], sem_s2).wait()

            def do_batch(b, nb, cur, oth):
                sidx, slocal, dlbuf, rows, sqbuf = cur
                pltpu.make_async_copy(b_hbm.at[sidx], rows, sem_g).wait()
                pltpu.async_copy(rows, ssum.at[slocal], sem_s, add=True)

                @pl.when(b + 1 < nb)
                def _():
                    @pl.when(b >= 1)
                    def _():
                        wait_sum2(oth)
                    fill_and_fire((b + 1) * BATCH, oth)

                def edge_body(i, carry2):
                    dl = dlbuf[pl.ds(i, 16)][0]
                    for c8 in range(8):
                        cs = pl.ds(c8 * 16, 16)
                        r = rows[i, cs]
                        sqbuf[i, cs] = r * r
                        mn_acc[dl, cs] = jnp.minimum(mn_acc[dl, cs], r)
                        mx_acc[dl, cs] = jnp.maximum(mx_acc[dl, cs], r)
                    return carry2
                lax.fori_loop(0, BATCH, edge_body, 0, unroll=2)
                pltpu.make_async_copy(rows, ssum.at[slocal], sem_s).wait()
                pltpu.async_copy(sqbuf, ssum2.at[slocal], sem_s2, add=True)

            def flush(nb):
                @pl.when(nb > 0)
                def _():
                    fill_and_fire(0, bufsA)

                def bloop(b, carry):
                    par2 = lax.rem(b, 2)

                    @pl.when(par2 == 0)
                    def _():
                        do_batch(b, nb, bufsA, bufsB)

                    @pl.when(par2 == 1)
                    def _():
                        do_batch(b, nb, bufsB, bufsA)
                    return carry
                lax.fori_loop(0, nb, bloop, 0)

                @pl.when(nb >= 2)
                def _():
                    @pl.when(lax.rem(nb, 2) == 0)
                    def _():
                        wait_sum2(bufsA)
                    @pl.when(lax.rem(nb, 2) == 1)
                    def _():
                        wait_sum2(bufsB)

                @pl.when(nb >= 1)
                def _():
                    @pl.when(lax.rem(nb, 2) == 1)
                    def _():
                        wait_sum2(bufsA)
                    @pl.when(lax.rem(nb, 2) == 0)
                    def _():
                        wait_sum2(bufsB)

            pltpu.async_copy(dst_hbm.at[pl.ds(0, CHUNK)],
                             dbuf.at[pl.ds(0, CHUNK)], sem_e)
            pltpu.async_copy(src_hbm.at[pl.ds(0, CHUNK)],
                             sbuf.at[pl.ds(0, CHUNK)], sem_e)

            def chunk_body(k, pos):
                par = lax.rem(k, 2)
                off0 = par * CHUNK
                pltpu.make_async_copy(dst_hbm.at[pl.ds(k * CHUNK, CHUNK)],
                                      dbuf.at[pl.ds(off0, CHUNK)], sem_e).wait()
                pltpu.make_async_copy(src_hbm.at[pl.ds(k * CHUNK, CHUNK)],
                                      sbuf.at[pl.ds(off0, CHUNK)], sem_e).wait()

                @pl.when(k < NCHUNK - 1)
                def _():
                    noff = (1 - par) * CHUNK
                    pltpu.async_copy(dst_hbm.at[pl.ds((k + 1) * CHUNK, CHUNK)],
                                     dbuf.at[pl.ds(noff, CHUNK)], sem_e)
                    pltpu.async_copy(src_hbm.at[pl.ds((k + 1) * CHUNK, CHUNK)],
                                     sbuf.at[pl.ds(noff, CHUNK)], sem_e)

                ones16 = jnp.ones((16,), jnp.int32)

                def scan_body(jj, p):
                    base = off0 + jj * 128
                    gs = []
                    for g in range(8):
                        d = dbuf[pl.ds(base + g * 16, 16)]
                        sv = sbuf[pl.ds(base + g * 16, 16)]
                        m = (d >= lo) & (d < lo + HRPT)
                        cs = plsc.cumsum(m.astype(jnp.int32))
                        gs.append((d, sv, m, cs, cs[15]))
                    pg = p
                    for d, sv, m, cs, cnt in gs:
                        posv = pg + cs - 1
                        plsc.store_scatter(mlist_d, [posv], d - (lo - sbase),
                                           mask=m)
                        plsc.store_scatter(mlist_s, [posv], sv, mask=m)
                        plsc.addupdate_scatter(cntv, [d - lo_t], ones16, mask=m)
                        pg = pg + cnt
                    return pg
                pos = lax.fori_loop(0, CHUNK // 128, scan_body, pos)

                nb = jnp.where(pos >= TFLUSH, pos // BATCH, 0)
                flush(nb)
                rem = pos - nb * BATCH

                @pl.when(nb > 0)
                def _():
                    for kk in range(8):
                        vd = mlist_d[pl.ds(nb * BATCH + kk * 16, 16)]
                        vs = mlist_s[pl.ds(nb * BATCH + kk * 16, 16)]
                        mlist_d[pl.ds(kk * 16, 16)] = vd
                        mlist_s[pl.ds(kk * 16, 16)] = vs
                return rem

            pos = lax.fori_loop(0, NCHUNK, chunk_body, jnp.int32(0))

            @pl.when(pos > 0)
            def _():
                fb = (pos // BATCH) * BATCH
                for kk in range(BATCH // 16):
                    gi = lax.iota(jnp.int32, 16) + (kk * 16) + fb
                    keep = gi < pos
                    vs = mlist_s[pl.ds(fb + kk * 16, 16)]
                    mlist_s[pl.ds(fb + kk * 16, 16)] = jnp.where(keep, vs, 0)
                    vd = mlist_d[pl.ds(fb + kk * 16, 16)]
                    mlist_d[pl.ds(fb + kk * 16, 16)] = jnp.where(keep, vd,
                                                                 STRASH)
                flush((pos + BATCH - 1) // BATCH)

            pltpu.sync_copy(mn_acc.at[pl.ds(0, HRPT)],
                            mn_out.at[pl.ds(lo, HRPT)])
            pltpu.sync_copy(mx_acc.at[pl.ds(0, HRPT)],
                            mx_out.at[pl.ds(lo, HRPT)])
            pltpu.sync_copy(ssum.at[pl.ds(sbase, HRPT)],
                            sum_out.at[pl.ds(lo, HRPT)])
            pltpu.sync_copy(ssum2.at[pl.ds(sbase, HRPT)],
                            sum2_out.at[pl.ds(lo, HRPT)])

        pltpu.sync_copy(cntv, cnt_out.at[wid])

    return sck(dst, src, B)


def _post_body(x_ref, a_ref, cnt_ref, sb_ref, sb2_ref, mn_ref, mx_ref,
               wp_ref, bp_ref, wl_ref, bl_ref, out_ref):
    x = x_ref[...]
    a = a_ref[...]
    cnt = cnt_ref[...].astype(jnp.float32)
    cntc = jnp.maximum(cnt, 1.0)
    inv = 1.0 / cntc
    has = cnt > 0.0
    frac = cnt * inv
    sb = sb_ref[...]
    mb = sb * inv
    mb2 = sb2_ref[...] * inv
    mean = a * frac + mb
    var = jax.nn.relu(mb2 - mb * mb)
    std = jnp.sqrt(var + 1e-5)
    mn = jnp.where(has, a + mn_ref[...], 0.0)
    mx = jnp.where(has, a + mx_ref[...], 0.0)
    logd = jnp.log(cntc + 1.0)
    amp = logd * (1.0 / AVG_LOG)
    att = AVG_LOG / logd
    parts = [x, mean, mn, mx, std,
             mean * amp, mn * amp, mx * amp, std * amp,
             mean * att, mn * att, mx * att, std * att]
    wp = wp_ref[...]
    dn = (((1,), (0,)), ((), ()))
    out1 = bp_ref[...].astype(jnp.float32)
    for i, p in enumerate(parts):
        out1 = out1 + jax.lax.dot_general(
            p, wp[i * F:(i + 1) * F, :], dn,
            preferred_element_type=jnp.float32,
            precision=jax.lax.Precision.HIGHEST)
    out2 = jax.lax.dot_general(out1, wl_ref[...], dn,
                               preferred_element_type=jnp.float32,
                               precision=jax.lax.Precision.HIGHEST)
    out_ref[...] = out2 + bl_ref[...]


def _post(x, A, cnt, sB, sB2, mnB, mxB, W_post, b_post, W_lin, b_lin):
    wp = W_post.T
    wl = W_lin.T
    blk = lambda i: (i, 0)
    full = lambda i: (0, 0)
    return pl.pallas_call(
        _post_body,
        grid=(N // BLK,),
        in_specs=[
            pl.BlockSpec((BLK, F), blk),
            pl.BlockSpec((BLK, F), blk),
            pl.BlockSpec((BLK, 1), blk),
            pl.BlockSpec((BLK, F), blk),
            pl.BlockSpec((BLK, F), blk),
            pl.BlockSpec((BLK, F), blk),
            pl.BlockSpec((BLK, F), blk),
            pl.BlockSpec((13 * F, F), full),
            pl.BlockSpec((1, F), full),
            pl.BlockSpec((F, F), full),
            pl.BlockSpec((1, F), full),
        ],
        out_specs=pl.BlockSpec((BLK, F), blk),
        out_shape=jax.ShapeDtypeStruct((N, F), jnp.float32),
    )(x, A, cnt.reshape(N, 1), sB, sB2, mnB, mxB,
      wp, b_post.reshape(1, F), wl, b_lin.reshape(1, F))


def kernel(x, edge_index, W_pre, b_pre, W_post, b_post, W_lin, b_lin):
    src = edge_index[0]
    dst = edge_index[1]
    AB = _pre(x, W_pre, b_pre)
    A = AB[:, :F]
    B = AB[:, F:]
    cnt2d, sB, sB2, mnB, mxB = _sc_segment_stats(dst, src, B)
    cnt = cnt2d.reshape(NPAD)[:N]
    return _post(x, A, cnt, sB[:N], sB2[:N], mnB[:N], mxB[:N],
                 W_post, b_post, W_lin, b_lin)
